# baseline (speedup 1.0000x reference)
import functools

import jax
import jax.numpy as jnp
from jax.experimental import pallas as pl
from jax.experimental.pallas import tpu as pltpu
from jax.experimental.pallas import tpu_sc as plsc

_B, _T, _D = 1, 2048, 1024
_E, _K = 8, 2
_H = _D * 4

_TB = 256
_BLK = 1024
_ROWS = _T * _K + _E * _BLK
_NBLK = _ROWS // _BLK

_NC, _NS = 2, 16
_NW = _NC * _NS
_CH = _T // _NW


def _router_kernel(x_ref, lns_ref, lnb_ref, rw_ref, rb_ref,
                   h_ref, d0_ref, d1_ref, gates_ref, counts_ref):
    x = x_ref[...]
    mean = jnp.mean(x, axis=-1, keepdims=True)
    xc = x - mean
    var = jnp.mean(xc * xc, axis=-1, keepdims=True)
    h = xc * jax.lax.rsqrt(var + 1e-5) * lns_ref[...] + lnb_ref[...]
    h_ref[...] = h

    logits = jnp.dot(h.astype(jnp.bfloat16), rw_ref[...].astype(jnp.bfloat16).T,
                     preferred_element_type=jnp.float32)
    logits = logits + rb_ref[...]

    ar = jax.lax.broadcasted_iota(jnp.int32, logits.shape, 1)
    big = jnp.int32(_E)
    v0 = jnp.max(logits, axis=-1, keepdims=True)
    i0 = jnp.min(jnp.where(logits == v0, ar, big), axis=-1, keepdims=True)
    oh0 = (ar == i0).astype(jnp.int32)
    l2 = jnp.where(ar == i0, -jnp.inf, logits)
    v1 = jnp.max(l2, axis=-1, keepdims=True)
    i1 = jnp.min(jnp.where(l2 == v1, ar, big), axis=-1, keepdims=True)
    oh1 = (ar == i1).astype(jnp.int32)

    t = jnp.exp(v1 - v0)
    g0 = 1.0 / (1.0 + t)
    g1 = t / (1.0 + t)
    gates_ref[...] = jnp.concatenate([g0, g1], axis=1)

    c = oh0 + oh1
    s = c
    sh = 1
    while sh < _T:
        pad = jnp.zeros((sh, _E), jnp.int32)
        s = s + jnp.concatenate([pad, s[:-sh]], axis=0)
        sh *= 2
    xp = s - c

    counts = s[_T - 1:_T, :]
    pc = ((counts + _BLK - 1) // _BLK) * _BLK
    o = pc
    lsh = 1
    while lsh < _E:
        zpad = jnp.zeros((1, lsh), jnp.int32)
        o = o + jnp.concatenate([zpad, o[:, :-lsh]], axis=1)
        lsh *= 2
    off = o - pc

    d0_ref[...] = jnp.sum(oh0 * (off + xp), axis=1, keepdims=True)
    d1_ref[...] = jnp.sum(oh1 * (off + xp), axis=1, keepdims=True)
    counts_ref[...] = counts


def _sc_scatter_body(h_hbm, d0_hbm, d1_hbm, xs_hbm,
                     idx0_v, idx1_v, rows_v, sem):
    wid = jax.lax.axis_index("s") * _NC + jax.lax.axis_index("c")
    base = wid * _CH
    pltpu.sync_copy(d0_hbm.at[pl.ds(base, _CH)], idx0_v)
    pltpu.sync_copy(d1_hbm.at[pl.ds(base, _CH)], idx1_v)
    pltpu.sync_copy(h_hbm.at[pl.ds(base, _CH)], rows_v)
    pltpu.async_copy(rows_v, xs_hbm.at[idx0_v], sem).wait()
    pltpu.async_copy(rows_v, xs_hbm.at[idx1_v], sem).wait()


def _sc_gather_body(y_hbm, d0_hbm, d1_hbm, yg0_hbm, yg1_hbm,
                    idx_v, rows_v, sem):
    wid = jax.lax.axis_index("s") * _NC + jax.lax.axis_index("c")
    base = wid * _CH
    pltpu.sync_copy(d0_hbm.at[pl.ds(base, _CH)], idx_v)
    pltpu.async_copy(y_hbm.at[idx_v], rows_v, sem).wait()
    pltpu.sync_copy(rows_v, yg0_hbm.at[pl.ds(base, _CH)])
    pltpu.sync_copy(d1_hbm.at[pl.ds(base, _CH)], idx_v)
    pltpu.async_copy(y_hbm.at[idx_v], rows_v, sem).wait()
    pltpu.sync_copy(rows_v, yg1_hbm.at[pl.ds(base, _CH)])


_HC = 8
_H2 = _H // _HC


def _ffn_kernel(nact_ref, be_ref, xs_ref, w1_ref, b1_ref, w2_ref, b2_ref,
                y_ref):
    hc = pl.program_id(1)

    @pl.when(pl.program_id(0) < nact_ref[0])
    def _():
        xb = xs_ref[...].astype(jnp.bfloat16)
        w1 = w1_ref[0].astype(jnp.bfloat16)
        hid = jnp.dot(xb, w1.T, preferred_element_type=jnp.float32)
        hid = hid + b1_ref[0]
        act = 0.5 * hid * (1.0 + jax.lax.erf(hid * 0.7071067811865476))
        w2 = w2_ref[0].astype(jnp.bfloat16)
        part = jnp.dot(act.astype(jnp.bfloat16), w2.T,
                       preferred_element_type=jnp.float32)

        @pl.when(hc == 0)
        def _():
            y_ref[...] = part + b2_ref[0]

        @pl.when(hc > 0)
        def _():
            y_ref[...] = y_ref[...] + part


def _combine_kernel(x_ref, g_ref, y0_ref, y1_ref, o_ref):
    g0 = g_ref[:, 0:1]
    g1 = g_ref[:, 1:2]
    o_ref[...] = x_ref[...] + g0 * y0_ref[...] + g1 * y1_ref[...]


def kernel(x, ln_scale, ln_bias, router_w, router_b, fc1_w, fc1_b, fc2_w, fc2_b):
    x2 = x.reshape(_T, _D)

    h, d0, d1, gates, counts = pl.pallas_call(
        _router_kernel,
        grid=(1,),
        in_specs=[
            pl.BlockSpec((_T, _D), lambda i: (0, 0)),
            pl.BlockSpec((1, _D), lambda i: (0, 0)),
            pl.BlockSpec((1, _D), lambda i: (0, 0)),
            pl.BlockSpec((_E, _D), lambda i: (0, 0)),
            pl.BlockSpec((1, _E), lambda i: (0, 0)),
        ],
        out_specs=[
            pl.BlockSpec((_T, _D), lambda i: (0, 0)),
            pl.BlockSpec((_T, 1), lambda i: (0, 0)),
            pl.BlockSpec((_T, 1), lambda i: (0, 0)),
            pl.BlockSpec((_T, _K), lambda i: (0, 0)),
            pl.BlockSpec((1, _E), lambda i: (0, 0)),
        ],
        out_shape=[
            jax.ShapeDtypeStruct((_T, _D), jnp.float32),
            jax.ShapeDtypeStruct((_T, 1), jnp.int32),
            jax.ShapeDtypeStruct((_T, 1), jnp.int32),
            jax.ShapeDtypeStruct((_T, _K), jnp.float32),
            jax.ShapeDtypeStruct((1, _E), jnp.int32),
        ],
    )(x2, ln_scale.reshape(1, _D), ln_bias.reshape(1, _D),
      router_w, router_b.reshape(1, _E))

    d0f = d0.reshape(_T)
    d1f = d1.reshape(_T)

    cnt = counts.reshape(_E)
    pcb = (cnt + _BLK - 1) // _BLK
    cum = jnp.cumsum(pcb)
    nact = cum[_E - 1:]
    ib = jnp.arange(_NBLK, dtype=jnp.int32)
    be_raw = jnp.sum((ib[:, None] >= cum[None, :]).astype(jnp.int32), axis=1)
    be = be_raw[jnp.minimum(ib, nact[0] - 1)]

    mesh = plsc.VectorSubcoreMesh(core_axis_name="c", subcore_axis_name="s",
                                  num_cores=_NC, num_subcores=_NS)

    xs = pl.kernel(
        _sc_scatter_body,
        out_type=jax.ShapeDtypeStruct((_ROWS, _D), jnp.float32),
        mesh=mesh,
        scratch_types=[
            pltpu.VMEM((_CH,), jnp.int32),
            pltpu.VMEM((_CH,), jnp.int32),
            pltpu.VMEM((_CH, _D), jnp.float32),
            pltpu.SemaphoreType.DMA,
        ],
    )(h, d0f, d1f)

    y = pl.pallas_call(
        _ffn_kernel,
        grid_spec=pltpu.PrefetchScalarGridSpec(
            num_scalar_prefetch=2,
            grid=(_NBLK, _HC),
            in_specs=[
                pl.BlockSpec((_BLK, _D),
                             lambda i, hc, na, bb: (jnp.minimum(i, na[0] - 1), 0)),
                pl.BlockSpec(
                    (1, _H2, _D),
                    lambda i, hc, na, bb:
                        (bb[i], jnp.where(i < na[0], hc, _HC - 1), 0)),
                pl.BlockSpec(
                    (1, 1, _H2),
                    lambda i, hc, na, bb:
                        (bb[i], 0, jnp.where(i < na[0], hc, _HC - 1))),
                pl.BlockSpec(
                    (1, _D, _H2),
                    lambda i, hc, na, bb:
                        (bb[i], 0, jnp.where(i < na[0], hc, _HC - 1))),
                pl.BlockSpec((1, 1, _D), lambda i, hc, na, bb: (bb[i], 0, 0)),
            ],
            out_specs=pl.BlockSpec(
                (_BLK, _D), lambda i, hc, na, bb: (jnp.minimum(i, na[0] - 1), 0)),
        ),
        out_shape=jax.ShapeDtypeStruct((_ROWS, _D), jnp.float32),
    )(nact, be, xs, fc1_w, fc1_b.reshape(_E, 1, _H), fc2_w,
      fc2_b.reshape(_E, 1, _D))

    yg0, yg1 = pl.kernel(
        _sc_gather_body,
        out_type=(
            jax.ShapeDtypeStruct((_T, _D), jnp.float32),
            jax.ShapeDtypeStruct((_T, _D), jnp.float32),
        ),
        mesh=mesh,
        scratch_types=[
            pltpu.VMEM((_CH,), jnp.int32),
            pltpu.VMEM((_CH, _D), jnp.float32),
            pltpu.SemaphoreType.DMA,
        ],
    )(y, d0f, d1f)

    out = pl.pallas_call(
        _combine_kernel,
        grid=(_T // _TB,),
        in_specs=[
            pl.BlockSpec((_TB, _D), lambda i: (i, 0)),
            pl.BlockSpec((_TB, _K), lambda i: (i, 0)),
            pl.BlockSpec((_TB, _D), lambda i: (i, 0)),
            pl.BlockSpec((_TB, _D), lambda i: (i, 0)),
        ],
        out_specs=pl.BlockSpec((_TB, _D), lambda i: (i, 0)),
        out_shape=jax.ShapeDtypeStruct((_T, _D), jnp.float32),
    )(x2, gates, yg0, yg1)

    return out.reshape(_B, _T, _D)


# speedup vs baseline: 1.1807x; 1.1807x over previous
import functools

import jax
import jax.numpy as jnp
from jax.experimental import pallas as pl
from jax.experimental.pallas import tpu as pltpu
from jax.experimental.pallas import tpu_sc as plsc

_B, _T, _D = 1, 2048, 1024
_E, _K = 8, 2
_H = _D * 4

_TB = 512
_BLK = 1024
_ROWS = _T * _K + _E * _BLK
_NBLK = _ROWS // _BLK

_NC, _NS = 2, 16
_NW = _NC * _NS
_CH = _T // _NW


def _router_kernel(x_ref, lns_ref, lnb_ref, rw_ref, rb_ref,
                   h_ref, d0_ref, d1_ref, gates_ref, be_ref, nact_ref):
    x = x_ref[...]
    mean = jnp.mean(x, axis=-1, keepdims=True)
    xc = x - mean
    var = jnp.mean(xc * xc, axis=-1, keepdims=True)
    h = xc * jax.lax.rsqrt(var + 1e-5) * lns_ref[...] + lnb_ref[...]
    h_ref[...] = h

    logits = jnp.dot(h.astype(jnp.bfloat16), rw_ref[...].astype(jnp.bfloat16).T,
                     preferred_element_type=jnp.float32)
    logits = logits + rb_ref[...]

    ar = jax.lax.broadcasted_iota(jnp.int32, logits.shape, 1)
    big = jnp.int32(_E)
    v0 = jnp.max(logits, axis=-1, keepdims=True)
    i0 = jnp.min(jnp.where(logits == v0, ar, big), axis=-1, keepdims=True)
    oh0 = (ar == i0).astype(jnp.int32)
    l2 = jnp.where(ar == i0, -jnp.inf, logits)
    v1 = jnp.max(l2, axis=-1, keepdims=True)
    i1 = jnp.min(jnp.where(l2 == v1, ar, big), axis=-1, keepdims=True)
    oh1 = (ar == i1).astype(jnp.int32)

    t = jnp.exp(v1 - v0)
    g0 = 1.0 / (1.0 + t)
    g1 = t / (1.0 + t)
    gates_ref[...] = jnp.concatenate([g0, g1], axis=1)

    c = oh0 + oh1
    s = c
    sh = 1
    while sh < _T:
        pad = jnp.zeros((sh, _E), jnp.int32)
        s = s + jnp.concatenate([pad, s[:-sh]], axis=0)
        sh *= 2
    xp = s - c

    counts = s[_T - 1:_T, :]
    pc = ((counts + _BLK - 1) // _BLK) * _BLK
    o = pc
    lsh = 1
    while lsh < _E:
        zpad = jnp.zeros((1, lsh), jnp.int32)
        o = o + jnp.concatenate([zpad, o[:, :-lsh]], axis=1)
        lsh *= 2
    off = o - pc

    d0_ref[...] = jnp.sum(oh0 * (off + xp), axis=1, keepdims=True)
    d1_ref[...] = jnp.sum(oh1 * (off + xp), axis=1, keepdims=True)

    cum = o
    cumb = cum // _BLK
    nact = cumb[:, _E - 1:_E]
    ibs = jax.lax.broadcasted_iota(jnp.int32, (_NBLK, _E), 0)
    be_raw = jnp.sum((ibs >= cumb).astype(jnp.int32), axis=1, keepdims=True)
    belast = jnp.sum(((nact - 1) >= cumb).astype(jnp.int32),
                     axis=1, keepdims=True)
    ib1 = jax.lax.broadcasted_iota(jnp.int32, (_NBLK, 1), 0)
    be_ref[...] = jnp.where(ib1 < nact, be_raw, belast)
    nact_ref[...] = nact


def _sc_scatter_body(h_hbm, d0_hbm, d1_hbm, xs_hbm,
                     idx0_v, idx1_v, rows_v, sem):
    wid = jax.lax.axis_index("s") * _NC + jax.lax.axis_index("c")
    base = wid * _CH
    pltpu.sync_copy(d0_hbm.at[pl.ds(base, _CH)], idx0_v)
    pltpu.sync_copy(d1_hbm.at[pl.ds(base, _CH)], idx1_v)
    pltpu.sync_copy(h_hbm.at[pl.ds(base, _CH)], rows_v)
    pltpu.async_copy(rows_v, xs_hbm.at[idx0_v], sem).wait()
    pltpu.async_copy(rows_v, xs_hbm.at[idx1_v], sem).wait()


def _sc_gather_body(y_hbm, d0_hbm, d1_hbm, yg0_hbm, yg1_hbm,
                    idx_v, rows_v, sem):
    wid = jax.lax.axis_index("s") * _NC + jax.lax.axis_index("c")
    base = wid * _CH
    pltpu.sync_copy(d0_hbm.at[pl.ds(base, _CH)], idx_v)
    pltpu.async_copy(y_hbm.at[idx_v], rows_v, sem).wait()
    pltpu.sync_copy(rows_v, yg0_hbm.at[pl.ds(base, _CH)])
    pltpu.sync_copy(d1_hbm.at[pl.ds(base, _CH)], idx_v)
    pltpu.async_copy(y_hbm.at[idx_v], rows_v, sem).wait()
    pltpu.sync_copy(rows_v, yg1_hbm.at[pl.ds(base, _CH)])


_HC = 4
_H2 = _H // _HC


def _ffn_kernel(nact_ref, be_ref, xs_ref, w1_ref, b1_ref, w2_ref, b2_ref,
                y_ref):
    hc = pl.program_id(1)

    @pl.when(pl.program_id(0) < nact_ref[0])
    def _():
        xb = xs_ref[...].astype(jnp.bfloat16)
        w1 = w1_ref[0].astype(jnp.bfloat16)
        hid = jnp.dot(xb, w1.T, preferred_element_type=jnp.float32)
        hid = hid + b1_ref[0]
        act = 0.5 * hid * (1.0 + jax.lax.erf(hid * 0.7071067811865476))
        w2 = w2_ref[0].astype(jnp.bfloat16)
        part = jnp.dot(act.astype(jnp.bfloat16), w2.T,
                       preferred_element_type=jnp.float32)

        @pl.when(hc == 0)
        def _():
            y_ref[...] = part + b2_ref[0]

        @pl.when(hc > 0)
        def _():
            y_ref[...] = y_ref[...] + part


def _combine_kernel(x_ref, g_ref, y0_ref, y1_ref, o_ref):
    g0 = g_ref[:, 0:1]
    g1 = g_ref[:, 1:2]
    o_ref[...] = x_ref[...] + g0 * y0_ref[...] + g1 * y1_ref[...]


def kernel(x, ln_scale, ln_bias, router_w, router_b, fc1_w, fc1_b, fc2_w, fc2_b):
    x2 = x.reshape(_T, _D)

    h, d0, d1, gates, be2, nact2 = pl.pallas_call(
        _router_kernel,
        grid=(1,),
        in_specs=[
            pl.BlockSpec((_T, _D), lambda i: (0, 0)),
            pl.BlockSpec((1, _D), lambda i: (0, 0)),
            pl.BlockSpec((1, _D), lambda i: (0, 0)),
            pl.BlockSpec((_E, _D), lambda i: (0, 0)),
            pl.BlockSpec((1, _E), lambda i: (0, 0)),
        ],
        out_specs=[
            pl.BlockSpec((_T, _D), lambda i: (0, 0)),
            pl.BlockSpec((_T, 1), lambda i: (0, 0)),
            pl.BlockSpec((_T, 1), lambda i: (0, 0)),
            pl.BlockSpec((_T, _K), lambda i: (0, 0)),
            pl.BlockSpec((_NBLK, 1), lambda i: (0, 0)),
            pl.BlockSpec((1, 1), lambda i: (0, 0)),
        ],
        out_shape=[
            jax.ShapeDtypeStruct((_T, _D), jnp.float32),
            jax.ShapeDtypeStruct((_T, 1), jnp.int32),
            jax.ShapeDtypeStruct((_T, 1), jnp.int32),
            jax.ShapeDtypeStruct((_T, _K), jnp.float32),
            jax.ShapeDtypeStruct((_NBLK, 1), jnp.int32),
            jax.ShapeDtypeStruct((1, 1), jnp.int32),
        ],
    )(x2, ln_scale.reshape(1, _D), ln_bias.reshape(1, _D),
      router_w, router_b.reshape(1, _E))

    d0f = d0.reshape(_T)
    d1f = d1.reshape(_T)
    be = be2.reshape(_NBLK)
    nact = nact2.reshape(1)

    mesh = plsc.VectorSubcoreMesh(core_axis_name="c", subcore_axis_name="s",
                                  num_cores=_NC, num_subcores=_NS)

    xs = pl.kernel(
        _sc_scatter_body,
        out_type=jax.ShapeDtypeStruct((_ROWS, _D), jnp.float32),
        mesh=mesh,
        scratch_types=[
            pltpu.VMEM((_CH,), jnp.int32),
            pltpu.VMEM((_CH,), jnp.int32),
            pltpu.VMEM((_CH, _D), jnp.float32),
            pltpu.SemaphoreType.DMA,
        ],
    )(h, d0f, d1f)

    y = pl.pallas_call(
        _ffn_kernel,
        grid_spec=pltpu.PrefetchScalarGridSpec(
            num_scalar_prefetch=2,
            grid=(_NBLK, _HC),
            in_specs=[
                pl.BlockSpec((_BLK, _D),
                             lambda i, hc, na, bb: (jnp.minimum(i, na[0] - 1), 0)),
                pl.BlockSpec(
                    (1, _H2, _D),
                    lambda i, hc, na, bb:
                        (bb[i], jnp.where(i < na[0], hc, _HC - 1), 0)),
                pl.BlockSpec(
                    (1, 1, _H2),
                    lambda i, hc, na, bb:
                        (bb[i], 0, jnp.where(i < na[0], hc, _HC - 1))),
                pl.BlockSpec(
                    (1, _D, _H2),
                    lambda i, hc, na, bb:
                        (bb[i], 0, jnp.where(i < na[0], hc, _HC - 1))),
                pl.BlockSpec((1, 1, _D), lambda i, hc, na, bb: (bb[i], 0, 0)),
            ],
            out_specs=pl.BlockSpec(
                (_BLK, _D), lambda i, hc, na, bb: (jnp.minimum(i, na[0] - 1), 0)),
        ),
        out_shape=jax.ShapeDtypeStruct((_ROWS, _D), jnp.float32),
    )(nact, be, xs, fc1_w, fc1_b.reshape(_E, 1, _H), fc2_w,
      fc2_b.reshape(_E, 1, _D))

    yg0, yg1 = pl.kernel(
        _sc_gather_body,
        out_type=(
            jax.ShapeDtypeStruct((_T, _D), jnp.float32),
            jax.ShapeDtypeStruct((_T, _D), jnp.float32),
        ),
        mesh=mesh,
        scratch_types=[
            pltpu.VMEM((_CH,), jnp.int32),
            pltpu.VMEM((_CH, _D), jnp.float32),
            pltpu.SemaphoreType.DMA,
        ],
    )(y, d0f, d1f)

    out = pl.pallas_call(
        _combine_kernel,
        grid=(_T // _TB,),
        in_specs=[
            pl.BlockSpec((_TB, _D), lambda i: (i, 0)),
            pl.BlockSpec((_TB, _K), lambda i: (i, 0)),
            pl.BlockSpec((_TB, _D), lambda i: (i, 0)),
            pl.BlockSpec((_TB, _D), lambda i: (i, 0)),
        ],
        out_specs=pl.BlockSpec((_TB, _D), lambda i: (i, 0)),
        out_shape=jax.ShapeDtypeStruct((_T, _D), jnp.float32),
    )(x2, gates, yg0, yg1)

    return out.reshape(_B, _T, _D)
